# baseline (speedup 1.0000x reference)
import functools

import jax
import jax.numpy as jnp
from jax import lax
from jax.experimental import pallas as pl
from jax.experimental.pallas import tpu as pltpu
from jax.experimental.pallas import tpu_sc as plsc

EMB = 64
SCALE = 8.0

NC = 2
NS = 16
NW = NC * NS

BW = 128


def _sc_gather(gidx2d, tabR):
    t_dim, b_dim = gidx2d.shape
    nbh = b_dim // BW
    nblk = t_dim * nbh
    blk_per_w = nblk // NW

    mesh = plsc.VectorSubcoreMesh(core_axis_name="c", subcore_axis_name="s")

    @functools.partial(
        pl.kernel,
        mesh=mesh,
        out_type=jax.ShapeDtypeStruct((t_dim, 8, nbh, 8 * BW), jnp.float32),
        scratch_types=[
            pltpu.VMEM((4, BW), jnp.int32),
            pltpu.VMEM((4, BW, EMB), jnp.float32),
            pltpu.VMEM((2 * 8 * 8 * BW,), jnp.float32),
            pltpu.SemaphoreType.DMA((4,)),
            pltpu.SemaphoreType.DMA((4,)),
            pltpu.SemaphoreType.DMA((2,)),
        ],
        compiler_params=pltpu.CompilerParams(use_tc_tiling_on_sc=False,
                                             needs_layout_passes=False),
    )
    def k(gidx_hbm, tab_hbm, out_hbm, idx_v, rows_v, tb_v, isem, gsem, ssem):
        wid = lax.axis_index("s") * NC + lax.axis_index("c")
        f0 = wid * blk_per_w
        iota16 = jnp.arange(16, dtype=jnp.int32)

        def tb_of(f):
            return f // nbh, f % nbh

        def fire_idx(f, b):
            t, bh = tb_of(f)
            pltpu.async_copy(gidx_hbm.at[t, pl.ds(bh * BW, BW)],
                             idx_v.at[b], isem.at[b])

        def wait_idx(b):
            pltpu.make_async_copy(gidx_hbm.at[0, pl.ds(0, BW)],
                                  idx_v.at[b], isem.at[b]).wait()

        def fire_gather(b):
            pltpu.async_copy(tab_hbm.at[idx_v.at[b]], rows_v.at[b],
                             gsem.at[b])

        def wait_gather(b):
            pltpu.make_async_copy(tab_hbm.at[pl.ds(0, BW)],
                                  rows_v.at[b], gsem.at[b]).wait()

        def fire_store(f, b):
            t, bh = tb_of(f)
            for ehi in range(8):
                pltpu.async_copy(tb_v.at[pl.ds(b * 8192 + ehi * 1024, 1024)],
                                 out_hbm.at[t, ehi, bh], ssem.at[b])

        def wait_store(b):
            for _ in range(8):
                pltpu.make_async_copy(tb_v.at[pl.ds(0, 1024)],
                                      out_hbm.at[0, 0, 0], ssem.at[b]).wait()

        i128 = iota16 * BW

        def transpose_scale(src_b, dst_b):
            dst_off = dst_b * 8192

            @plsc.parallel_loop(0, BW, step=1, unroll=4)
            def _(lane):
                for e0 in range(0, EMB, 16):
                    v = rows_v[src_b, lane, pl.ds(e0, 16)]
                    addr = i128 + (lane + (dst_off + e0 * BW))
                    plsc.store_scatter(tb_v, [addr], v * SCALE)

        for j in range(4):
            fire_idx(f0 + j, j)
        for j in range(3):
            wait_idx(j)
            fire_gather(j)

        def step(i, carry):
            for b4 in range(4):
                g = i * 4 + b4
                b2 = b4 % 2
                wait_gather(b4)
                transpose_scale(b4, b2)

                @pl.when(g >= 2)
                def _():
                    wait_store(b2)

                fire_store(f0 + g, b2)

                @pl.when(g + 3 < blk_per_w)
                def _():
                    wait_idx((g + 3) % 4)
                    fire_gather((g + 3) % 4)

                @pl.when(g + 4 < blk_per_w)
                def _():
                    fire_idx(f0 + g + 4, b4)

            return carry

        lax.fori_loop(0, blk_per_w // 4, step, 0)
        wait_store(0)
        wait_store(1)

    return k(gidx2d, tabR)


def kernel(tokens, table):
    b0, b1 = tokens.shape
    tokT = jnp.swapaxes(tokens, 0, 1).astype(jnp.int32)
    outv = _sc_gather(tokT, table)
    outv = jnp.reshape(outv, (b1, 8, b0 // BW, 8, BW))
    r = jnp.transpose(outv, (0, 1, 3, 2, 4))
    r = jnp.reshape(r, (b1, EMB, b0))
    return jnp.transpose(r, (2, 0, 1))


# speedup vs baseline: 1.2436x; 1.2436x over previous
import functools

import jax
import jax.numpy as jnp
from jax import lax
from jax.experimental import pallas as pl
from jax.experimental.pallas import tpu as pltpu
from jax.experimental.pallas import tpu_sc as plsc

EMB = 64
SCALE = 8.0

NC = 2
NS = 16
NW = NC * NS

BW = 128

CB = 2048


def _tc_relayout(tabT):
    e_dim, v_dim = tabT.shape
    ngrid = (v_dim + CB - 1) // CB

    def body(x_ref, o_ref):
        xt = jnp.swapaxes(x_ref[...], 0, 1)
        o_ref[:, :EMB] = xt[: CB // 2] * SCALE
        o_ref[:, EMB:] = xt[CB // 2 :] * SCALE

    return pl.pallas_call(
        body,
        grid=(ngrid,),
        in_specs=[pl.BlockSpec((e_dim, CB), lambda i: (0, i))],
        out_specs=pl.BlockSpec((CB // 2, 2 * EMB), lambda i: (i, 0)),
        out_shape=jax.ShapeDtypeStruct((ngrid * (CB // 2), 2 * EMB),
                                       jnp.float32),
    )(tabT)


def _sc_gather(gidx2d, tabR):
    t_dim, b_dim = gidx2d.shape
    nbh = b_dim // BW
    nblk = t_dim * nbh
    blk_per_w = nblk // NW

    mesh = plsc.VectorSubcoreMesh(core_axis_name="c", subcore_axis_name="s")

    @functools.partial(
        pl.kernel,
        mesh=mesh,
        out_type=jax.ShapeDtypeStruct((nblk * BW, EMB), jnp.float32),
        scratch_types=[
            pltpu.VMEM((8, BW), jnp.int32),
            pltpu.VMEM((8, BW, EMB), jnp.float32),
            pltpu.SemaphoreType.DMA((8,)),
            pltpu.SemaphoreType.DMA((8,)),
            pltpu.SemaphoreType.DMA((8,)),
        ],
        compiler_params=pltpu.CompilerParams(use_tc_tiling_on_sc=False,
                                             needs_layout_passes=False),
    )
    def k(gidx_hbm, tab_hbm, out_hbm, idx_v, rows_v, isem, gsem, ssem):
        wid = lax.axis_index("s") * NC + lax.axis_index("c")
        f0 = wid * blk_per_w

        def fire_idx(f, b):
            t = f // nbh
            bh = f - t * nbh
            pltpu.async_copy(gidx_hbm.at[t, pl.ds(bh * BW, BW)],
                             idx_v.at[b], isem.at[b])

        def wait_idx(b):
            pltpu.make_async_copy(gidx_hbm.at[0, pl.ds(0, BW)],
                                  idx_v.at[b], isem.at[b]).wait()
            for kk in range(BW // 16):
                v = idx_v[b, pl.ds(kk * 16, 16)]
                r = ((v & -2048) + ((v & 1023) << 1)) + ((v >> 10) & 1)
                idx_v[b, pl.ds(kk * 16, 16)] = r

        def fire_gather(b):
            pltpu.async_copy(tab_hbm.at[idx_v.at[b]], rows_v.at[b],
                             gsem.at[b])

        def wait_gather(b):
            pltpu.make_async_copy(tab_hbm.at[pl.ds(0, BW)],
                                  rows_v.at[b], gsem.at[b]).wait()

        def fire_store(f, b):
            pltpu.async_copy(rows_v.at[b],
                             out_hbm.at[pl.ds(f * BW, BW)], ssem.at[b])

        def wait_store(b):
            pltpu.make_async_copy(rows_v.at[b],
                                  out_hbm.at[pl.ds(0, BW)], ssem.at[b]).wait()

        for j in range(8):
            fire_idx(f0 + j, j)
        for j in range(4):
            wait_idx(j)
            fire_gather(j)

        def step(i, carry):
            for b8 in range(8):
                g = i * 8 + b8
                wait_gather(b8)
                fire_store(f0 + g, b8)
                n = g + 4

                @pl.when(n < blk_per_w)
                def _():
                    nb = (b8 + 4) % 8
                    wait_idx(nb)

                    @pl.when(n >= 8)
                    def _():
                        wait_store(nb)

                    fire_gather(nb)

                    @pl.when(n + 4 < blk_per_w)
                    def _():
                        fire_idx(f0 + n + 4, b8)

            return carry

        lax.fori_loop(0, blk_per_w // 8, step, 0)
        for b in range(8):
            wait_store(b)

    return k(gidx2d, tabR)


def kernel(tokens, table):
    b0, b1 = tokens.shape
    tokT = jnp.swapaxes(tokens, 0, 1).astype(jnp.int32)
    tabT = jnp.swapaxes(table, 0, 1)
    tabP = _tc_relayout(tabT)
    tabR = jnp.reshape(jnp.reshape(tabP, (-1,)),
                       (tabP.shape[0] * 2, EMB))
    outv = _sc_gather(tokT, tabR)
    o = jnp.reshape(outv, (b1, b0 // BW, BW, EMB))
    return jnp.reshape(jnp.transpose(o, (1, 2, 0, 3)), (b0, b1, EMB))
